# baseline (speedup 1.0000x reference)
import functools

import jax
import jax.numpy as jnp
import numpy as np
from jax import lax
from jax.experimental import pallas as pl
from jax.experimental.pallas import tpu as pltpu
from jax.experimental.pallas import tpu_sc as plsc

_NC = 2
_NS = 16
_NW = _NC * _NS

_I32_MIN = np.int32(-2147483648)


def _router_body(x_ref, rw_ref, o_ref):
    o_ref[...] = jnp.dot(x_ref[...], rw_ref[...],
                         preferred_element_type=jnp.float32)


def _router(xf, router_w):
    n, d = xf.shape
    tm = 512
    return pl.pallas_call(
        _router_body,
        grid=(n // tm,),
        in_specs=[
            pl.BlockSpec((tm, d), lambda i: (i, 0)),
            pl.BlockSpec((d, 1), lambda i: (0, 0)),
        ],
        out_specs=pl.BlockSpec((tm, 1), lambda i: (i, 0)),
        out_shape=jax.ShapeDtypeStruct((n, 1), jnp.float32),
    )(xf, router_w)


def _cumsum_col(v, s):
    sh = 1
    while sh < s:
        v = v + jnp.concatenate(
            [jnp.zeros((sh, 1), jnp.float32), v[:-sh, :]], axis=0)
        sh *= 2
    return v


def _route_body(k, w_ref, rb_ref, idx_ref, scale_ref, gate_ref):
    s = w_ref.shape[1]
    w = w_ref[0] + rb_ref[0, 0]
    bits = lax.bitcast_convert_type(w, jnp.int32)
    key = jnp.where(bits >= 0, bits, jnp.bitwise_xor(~bits, _I32_MIN))

    cnt = jnp.sum(jnp.where(key >= 0, 1, 0))
    cs = jnp.where(cnt >= k, jnp.int32(0), _I32_MIN)
    for t in range(30, -1, -1):
        test = jnp.bitwise_or(cs, jnp.int32(1 << t))
        cnt = jnp.sum(jnp.where(key >= test, 1, 0))
        cs = jnp.where(cnt >= k, test, cs)
    thr_bits = jnp.where(cs >= 0, cs, ~jnp.bitwise_xor(cs, _I32_MIN))
    thr = lax.bitcast_convert_type(thr_bits, jnp.float32)

    gt = (w > thr).astype(jnp.float32)
    eq = (w == thr).astype(jnp.float32)
    m = jnp.sum(gt)
    c2 = _cumsum_col(eq, s)
    fill = eq * jnp.where(m + c2 <= k, 1.0, 0.0)
    mask2 = gt + fill
    pos = _cumsum_col(mask2, s)

    jrow = lax.broadcasted_iota(jnp.int32, (1, k), 1).astype(jnp.float32) + 1.0
    one_hot = jnp.where(pos == jrow, 1.0, 0.0) * mask2
    iota_row = lax.broadcasted_iota(jnp.int32, (1, s), 1).astype(jnp.float32)
    idx_f = jnp.dot(iota_row, one_hot,
                    preferred_element_type=jnp.float32,
                    precision=lax.Precision.HIGHEST)
    b = pl.program_id(0)
    idx_ref[0, ...] = idx_f.astype(jnp.int32) + b * s

    scale_ref[0] = w * gt
    gate_ref[0] = gt


def _route(wts3, rb, k):
    bsz, s, _ = wts3.shape
    return pl.pallas_call(
        functools.partial(_route_body, k),
        grid=(bsz,),
        in_specs=[
            pl.BlockSpec((1, s, 1), lambda b: (b, 0, 0)),
            pl.BlockSpec((1, 1), lambda b: (0, 0)),
        ],
        out_specs=[
            pl.BlockSpec((1, 1, k), lambda b: (b, 0, 0)),
            pl.BlockSpec((1, s, 1), lambda b: (b, 0, 0)),
            pl.BlockSpec((1, s, 1), lambda b: (b, 0, 0)),
        ],
        out_shape=[
            jax.ShapeDtypeStruct((bsz, 1, k), jnp.int32),
            jax.ShapeDtypeStruct((bsz, s, 1), jnp.float32),
            jax.ShapeDtypeStruct((bsz, s, 1), jnp.float32),
        ],
    )(wts3, rb)


def _gather(xf, sg, gidx):
    n, d = xf.shape
    bk = gidx.shape[0]
    rpw = bk // _NW
    mesh = plsc.VectorSubcoreMesh(core_axis_name="c", subcore_axis_name="s",
                                  num_cores=_NC, num_subcores=_NS)

    @functools.partial(
        pl.kernel,
        out_type=[jax.ShapeDtypeStruct((bk, d), jnp.float32),
                  jax.ShapeDtypeStruct((bk, 128), jnp.float32)],
        mesh=mesh,
        scratch_types=[pltpu.VMEM((rpw,), jnp.int32),
                       pltpu.VMEM((rpw, d), jnp.float32),
                       pltpu.VMEM((rpw, 128), jnp.float32),
                       pltpu.SemaphoreType.DMA,
                       pltpu.SemaphoreType.DMA],
    )
    def _gather_k(x_hbm, sg_hbm, idx_hbm, xg_hbm, sgg_hbm,
                  idx_v, rows_v, sgv, sem1, sem2):
        wid = lax.axis_index("s") * _NC + lax.axis_index("c")
        base = wid * rpw
        pltpu.sync_copy(idx_hbm.at[pl.ds(base, rpw)], idx_v)
        c1 = pltpu.async_copy(x_hbm.at[idx_v], rows_v, sem1)
        c2 = pltpu.async_copy(sg_hbm.at[idx_v], sgv, sem2)
        c1.wait()
        c2.wait()
        pltpu.sync_copy(rows_v, xg_hbm.at[pl.ds(base, rpw)])
        pltpu.sync_copy(sgv, sgg_hbm.at[pl.ds(base, rpw)])

    return _gather_k(xf, sg, gidx)


def _mlp_body(xg_ref, w1_ref, b1_ref, w2_ref, b2_ref, sgg_ref, o_ref):
    j = pl.program_id(0)
    h = jnp.dot(xg_ref[...], w1_ref[...],
                preferred_element_type=jnp.float32) + b1_ref[...]
    h = jax.nn.gelu(h)
    part = jnp.dot(h, w2_ref[...], preferred_element_type=jnp.float32)

    @pl.when(j == 0)
    def _():
        o_ref[...] = part

    @pl.when(j > 0)
    def _():
        o_ref[...] += part

    @pl.when(j == pl.num_programs(0) - 1)
    def _():
        scale = sgg_ref[:, 0:1]
        gate = sgg_ref[:, 1:2]
        o_ref[...] = ((o_ref[...] + b2_ref[...]) * scale
                      + xg_ref[...] * (1.0 - gate))


def _mlp(xg, w1, b1r, w2, b2r, sgg):
    bk, d = xg.shape
    dff = w1.shape[1]
    tk = 512
    return pl.pallas_call(
        _mlp_body,
        grid=(dff // tk,),
        in_specs=[
            pl.BlockSpec((bk, d), lambda j: (0, 0)),
            pl.BlockSpec((d, tk), lambda j: (0, j)),
            pl.BlockSpec((1, tk), lambda j: (0, j)),
            pl.BlockSpec((tk, d), lambda j: (j, 0)),
            pl.BlockSpec((1, d), lambda j: (0, 0)),
            pl.BlockSpec((bk, 128), lambda j: (0, 0)),
        ],
        out_specs=pl.BlockSpec((bk, d), lambda j: (0, 0)),
        out_shape=jax.ShapeDtypeStruct((bk, d), jnp.float32),
    )(xg, w1, b1r, w2, b2r, sgg)


def _scatter(xf, vals, gidx):
    n, d = xf.shape
    bk = gidx.shape[0]
    cpw = n // _NW
    spw = bk // _NW
    mesh = plsc.VectorSubcoreMesh(core_axis_name="c", subcore_axis_name="s",
                                  num_cores=_NC, num_subcores=_NS)

    @functools.partial(
        pl.kernel,
        out_type=jax.ShapeDtypeStruct((n, d), jnp.float32),
        mesh=mesh,
        scratch_types=[pltpu.VMEM((spw,), jnp.int32),
                       pltpu.VMEM((spw, d), jnp.float32),
                       pltpu.SemaphoreType.DMA],
    )
    def _scatter_k(x_hbm, vals_hbm, idx_hbm, out_hbm, idx_v, rows_v, sem):
        cid = lax.axis_index("c")
        sid = lax.axis_index("s")
        wid = cid * _NS + sid
        pltpu.sync_copy(x_hbm.at[pl.ds(wid * cpw, cpw)],
                        out_hbm.at[pl.ds(wid * cpw, cpw)])
        plsc.subcore_barrier()
        p0 = wid * spw
        pltpu.sync_copy(idx_hbm.at[pl.ds(p0, spw)], idx_v)
        pltpu.sync_copy(vals_hbm.at[pl.ds(p0, spw)], rows_v)
        pltpu.async_copy(rows_v, out_hbm.at[idx_v], sem).wait()

    return _scatter_k(xf, vals, gidx)


def kernel(x, causal_mask, position_ids, router_w, router_b, w1, b1, w2, b2):
    b, s, d = x.shape
    dff = w1.shape[1]
    k = s // 8
    xf = x.reshape(b * s, d)

    wts = _router(xf, router_w)
    idx3, scale3, gate3 = _route(wts.reshape(b, s, 1),
                                 router_b.reshape(1, 1), k)
    gidx = idx3.reshape(b * k)

    sg = jnp.concatenate(
        [scale3.reshape(b * s, 1), gate3.reshape(b * s, 1),
         jnp.zeros((b * s, 126), jnp.float32)], axis=1)

    xg, sgg = _gather(xf, sg, gidx)
    vals = _mlp(xg, w1, b1.reshape(1, dff), w2, b2.reshape(1, d), sgg)
    out = _scatter(xf, vals, gidx)
    return (out.reshape(b, s, d),)


# speedup vs baseline: 2.3902x; 2.3902x over previous
import functools

import jax
import jax.numpy as jnp
import numpy as np
from jax import lax
from jax.experimental import pallas as pl
from jax.experimental.pallas import tpu as pltpu
from jax.experimental.pallas import tpu_sc as plsc

_NC = 2
_NS = 16
_NW = _NC * _NS

_I32_MIN = np.int32(-2147483648)


def _router_body(x_ref, rw_ref, o_ref):
    o_ref[...] = jnp.dot(x_ref[...], rw_ref[...],
                         preferred_element_type=jnp.float32)


def _router(xf, router_w):
    n, d = xf.shape
    tm = 512
    return pl.pallas_call(
        _router_body,
        grid=(n // tm,),
        in_specs=[
            pl.BlockSpec((tm, d), lambda i: (i, 0)),
            pl.BlockSpec((d, 1), lambda i: (0, 0)),
        ],
        out_specs=pl.BlockSpec((tm, 1), lambda i: (i, 0)),
        out_shape=jax.ShapeDtypeStruct((n, 1), jnp.float32),
    )(xf, router_w)


def _cumsum_col(v, s):
    sh = 1
    while sh < s:
        v = v + jnp.concatenate(
            [jnp.zeros((sh, 1), jnp.float32), v[:-sh, :]], axis=0)
        sh *= 2
    return v


def _route_body(k, w_ref, rb_ref, idx_ref, scale_ref, gate_ref):
    s = w_ref.shape[1]
    w = w_ref[0] + rb_ref[0, 0]
    bits = lax.bitcast_convert_type(w, jnp.int32)
    key = jnp.where(bits >= 0, bits, jnp.bitwise_xor(~bits, _I32_MIN))

    cnt = jnp.sum(jnp.where(key >= 0, 1, 0))
    cs = jnp.where(cnt >= k, jnp.int32(0), _I32_MIN)
    for t in range(30, -1, -1):
        test = jnp.bitwise_or(cs, jnp.int32(1 << t))
        cnt = jnp.sum(jnp.where(key >= test, 1, 0))
        cs = jnp.where(cnt >= k, test, cs)
    thr_bits = jnp.where(cs >= 0, cs, ~jnp.bitwise_xor(cs, _I32_MIN))
    thr = lax.bitcast_convert_type(thr_bits, jnp.float32)

    gt = (w > thr).astype(jnp.float32)
    eq = (w == thr).astype(jnp.float32)
    m = jnp.sum(gt)
    c2 = _cumsum_col(eq, s)
    fill = eq * jnp.where(m + c2 <= k, 1.0, 0.0)
    mask2 = gt + fill
    pos = _cumsum_col(mask2, s)

    jrow = lax.broadcasted_iota(jnp.int32, (1, k), 1).astype(jnp.float32) + 1.0
    one_hot = jnp.where(pos == jrow, 1.0, 0.0) * mask2
    iota_row = lax.broadcasted_iota(jnp.int32, (1, s), 1).astype(jnp.float32)
    idx_f = jnp.dot(iota_row, one_hot,
                    preferred_element_type=jnp.float32,
                    precision=lax.Precision.HIGHEST)
    b = pl.program_id(0)
    idx_ref[0, ...] = idx_f.astype(jnp.int32) + b * s

    scale_ref[0] = w * gt
    gate_ref[0] = gt


def _route(wts3, rb, k):
    bsz, s, _ = wts3.shape
    return pl.pallas_call(
        functools.partial(_route_body, k),
        grid=(bsz,),
        in_specs=[
            pl.BlockSpec((1, s, 1), lambda b: (b, 0, 0)),
            pl.BlockSpec((1, 1), lambda b: (0, 0)),
        ],
        out_specs=[
            pl.BlockSpec((1, 1, k), lambda b: (b, 0, 0)),
            pl.BlockSpec((1, s, 1), lambda b: (b, 0, 0)),
            pl.BlockSpec((1, s, 1), lambda b: (b, 0, 0)),
        ],
        out_shape=[
            jax.ShapeDtypeStruct((bsz, 1, k), jnp.int32),
            jax.ShapeDtypeStruct((bsz, s, 1), jnp.float32),
            jax.ShapeDtypeStruct((bsz, s, 1), jnp.float32),
        ],
    )(wts3, rb)


def _gather(xf, sg, gidx):
    n, d = xf.shape
    bk = gidx.shape[0]
    rpw = bk // _NW
    mesh = plsc.VectorSubcoreMesh(core_axis_name="c", subcore_axis_name="s",
                                  num_cores=_NC, num_subcores=_NS)

    @functools.partial(
        pl.kernel,
        out_type=[jax.ShapeDtypeStruct((bk, d), jnp.float32),
                  jax.ShapeDtypeStruct((bk, 128), jnp.float32)],
        mesh=mesh,
        scratch_types=[pltpu.VMEM((rpw,), jnp.int32),
                       pltpu.VMEM((rpw, d), jnp.float32),
                       pltpu.VMEM((rpw, 128), jnp.float32),
                       pltpu.SemaphoreType.DMA,
                       pltpu.SemaphoreType.DMA],
    )
    def _gather_k(x_hbm, sg_hbm, idx_hbm, xg_hbm, sgg_hbm,
                  idx_v, rows_v, sgv, sem1, sem2):
        wid = lax.axis_index("s") * _NC + lax.axis_index("c")
        base = wid * rpw
        pltpu.sync_copy(idx_hbm.at[pl.ds(base, rpw)], idx_v)
        c1 = pltpu.async_copy(x_hbm.at[idx_v], rows_v, sem1)
        c2 = pltpu.async_copy(sg_hbm.at[idx_v], sgv, sem2)
        c1.wait()
        c2.wait()
        pltpu.sync_copy(rows_v, xg_hbm.at[pl.ds(base, rpw)])
        pltpu.sync_copy(sgv, sgg_hbm.at[pl.ds(base, rpw)])

    return _gather_k(xf, sg, gidx)


def _mlp_body(xg_ref, w1_ref, b1_ref, w2_ref, b2_ref, sgg_ref, o_ref):
    j = pl.program_id(0)
    h = jnp.dot(xg_ref[...], w1_ref[...],
                preferred_element_type=jnp.float32) + b1_ref[...]
    h = jax.nn.gelu(h)
    part = jnp.dot(h, w2_ref[...], preferred_element_type=jnp.float32)

    @pl.when(j == 0)
    def _():
        o_ref[...] = part

    @pl.when(j > 0)
    def _():
        o_ref[...] += part

    @pl.when(j == pl.num_programs(0) - 1)
    def _():
        scale = sgg_ref[:, 0:1]
        gate = sgg_ref[:, 1:2]
        o_ref[...] = ((o_ref[...] + b2_ref[...]) * scale
                      + xg_ref[...] * (1.0 - gate))


def _mlp(xg, w1, b1r, w2, b2r, sgg):
    bk, d = xg.shape
    dff = w1.shape[1]
    tk = 512
    return pl.pallas_call(
        _mlp_body,
        grid=(dff // tk,),
        in_specs=[
            pl.BlockSpec((bk, d), lambda j: (0, 0)),
            pl.BlockSpec((d, tk), lambda j: (0, j)),
            pl.BlockSpec((1, tk), lambda j: (0, j)),
            pl.BlockSpec((tk, d), lambda j: (j, 0)),
            pl.BlockSpec((1, d), lambda j: (0, 0)),
            pl.BlockSpec((bk, 128), lambda j: (0, 0)),
        ],
        out_specs=pl.BlockSpec((bk, d), lambda j: (0, 0)),
        out_shape=jax.ShapeDtypeStruct((bk, d), jnp.float32),
    )(xg, w1, b1r, w2, b2r, sgg)


def _scatter_body(idx_ref, x_ref, v_ref, o_ref):
    del idx_ref, x_ref
    o_ref[...] = v_ref[...]


def _scatter(xf, vals, gidx):
    n, d = xf.shape
    bk = gidx.shape[0]
    grid_spec = pltpu.PrefetchScalarGridSpec(
        num_scalar_prefetch=1,
        grid=(bk,),
        in_specs=[
            pl.BlockSpec(memory_space=pl.ANY),
            pl.BlockSpec((1, 1, d), lambda i, idx: (i, 0, 0)),
        ],
        out_specs=pl.BlockSpec((1, 1, d), lambda i, idx: (idx[i], 0, 0)),
    )
    out = pl.pallas_call(
        _scatter_body,
        grid_spec=grid_spec,
        out_shape=jax.ShapeDtypeStruct((n, 1, d), jnp.float32),
        input_output_aliases={1: 0},
    )(gidx, xf.reshape(n, 1, d), vals.reshape(bk, 1, d))
    return out.reshape(n, d)


def kernel(x, causal_mask, position_ids, router_w, router_b, w1, b1, w2, b2):
    b, s, d = x.shape
    dff = w1.shape[1]
    k = s // 8
    xf = x.reshape(b * s, d)

    wts = _router(xf, router_w)
    idx3, scale3, gate3 = _route(wts.reshape(b, s, 1),
                                 router_b.reshape(1, 1), k)
    gidx = idx3.reshape(b * k)

    sg = jnp.concatenate(
        [scale3.reshape(b * s, 1), gate3.reshape(b * s, 1),
         jnp.zeros((b * s, 126), jnp.float32)], axis=1)

    xg, sgg = _gather(xf, sg, gidx)
    vals = _mlp(xg, w1, b1.reshape(1, dff), w2, b2.reshape(1, d), sgg)
    out = _scatter(xf, vals, gidx)
    return (out.reshape(b, s, d),)


# speedup vs baseline: 6.6659x; 2.7889x over previous
import functools

import jax
import jax.numpy as jnp
import numpy as np
from jax import lax
from jax.experimental import pallas as pl
from jax.experimental.pallas import tpu as pltpu
from jax.experimental.pallas import tpu_sc as plsc

_NC = 2
_NS = 16
_NW = _NC * _NS

_I32_MIN = np.int32(-2147483648)


def _router_body(x_ref, rw_ref, o_ref):
    o_ref[...] = jnp.dot(x_ref[...], rw_ref[...],
                         preferred_element_type=jnp.float32)


def _router(xf, router_w):
    n, d = xf.shape
    tm = 1024
    return pl.pallas_call(
        _router_body,
        grid=(n // tm,),
        in_specs=[
            pl.BlockSpec((tm, d), lambda i: (i, 0)),
            pl.BlockSpec((d, 1), lambda i: (0, 0)),
        ],
        out_specs=pl.BlockSpec((tm, 1), lambda i: (i, 0)),
        out_shape=jax.ShapeDtypeStruct((n, 1), jnp.float32),
    )(xf, router_w)


def _cumsum_col(v, s):
    sh = 1
    while sh < s:
        v = v + jnp.concatenate(
            [jnp.zeros((sh, 1), jnp.float32), v[:-sh, :]], axis=0)
        sh *= 2
    return v


def _route_body(k, w_ref, rb_ref, idx_ref, scale_ref, gate_ref):
    s = w_ref.shape[1]
    w = w_ref[0] + rb_ref[0, 0]
    bits = lax.bitcast_convert_type(w, jnp.int32)
    key = jnp.where(bits >= 0, bits, jnp.bitwise_xor(~bits, _I32_MIN))

    cnt = jnp.sum(jnp.where(key >= 0, 1, 0))
    cs = jnp.where(cnt >= k, jnp.int32(0), _I32_MIN)
    for t in range(30, -1, -1):
        test = jnp.bitwise_or(cs, jnp.int32(1 << t))
        cnt = jnp.sum(jnp.where(key >= test, 1, 0))
        cs = jnp.where(cnt >= k, test, cs)
    thr_bits = jnp.where(cs >= 0, cs, ~jnp.bitwise_xor(cs, _I32_MIN))
    thr = lax.bitcast_convert_type(thr_bits, jnp.float32)

    gt = (w > thr).astype(jnp.float32)
    eq = (w == thr).astype(jnp.float32)
    m = jnp.sum(gt)
    c2 = _cumsum_col(eq, s)
    fill = eq * jnp.where(m + c2 <= k, 1.0, 0.0)
    mask2 = gt + fill
    pos = _cumsum_col(mask2, s)

    jrow = lax.broadcasted_iota(jnp.int32, (1, k), 1).astype(jnp.float32) + 1.0
    one_hot = jnp.where(pos == jrow, 1.0, 0.0) * mask2
    iota_row = lax.broadcasted_iota(jnp.int32, (1, s), 1).astype(jnp.float32)
    idx_f = jnp.dot(iota_row, one_hot,
                    preferred_element_type=jnp.float32,
                    precision=lax.Precision.HIGHEST)
    b = pl.program_id(0)
    idx_ref[0, ...] = idx_f.astype(jnp.int32) + b * s

    scale_ref[0] = w * gt
    gate_ref[0] = gt


def _route(wts3, rb, k):
    bsz, s, _ = wts3.shape
    return pl.pallas_call(
        functools.partial(_route_body, k),
        grid=(bsz,),
        in_specs=[
            pl.BlockSpec((1, s, 1), lambda b: (b, 0, 0)),
            pl.BlockSpec((1, 1), lambda b: (0, 0)),
        ],
        out_specs=[
            pl.BlockSpec((1, 1, k), lambda b: (b, 0, 0)),
            pl.BlockSpec((1, s, 1), lambda b: (b, 0, 0)),
            pl.BlockSpec((1, s, 1), lambda b: (b, 0, 0)),
        ],
        out_shape=[
            jax.ShapeDtypeStruct((bsz, 1, k), jnp.int32),
            jax.ShapeDtypeStruct((bsz, s, 1), jnp.float32),
            jax.ShapeDtypeStruct((bsz, s, 1), jnp.float32),
        ],
    )(wts3, rb)


def _gather(xf, sg, gidx):
    n, d = xf.shape
    bk = gidx.shape[0]
    rpw = bk // _NW
    mesh = plsc.VectorSubcoreMesh(core_axis_name="c", subcore_axis_name="s",
                                  num_cores=_NC, num_subcores=_NS)

    @functools.partial(
        pl.kernel,
        out_type=[jax.ShapeDtypeStruct((bk, d), jnp.float32),
                  jax.ShapeDtypeStruct((bk, 128), jnp.float32)],
        mesh=mesh,
        scratch_types=[pltpu.VMEM((rpw,), jnp.int32),
                       pltpu.VMEM((rpw, d), jnp.float32),
                       pltpu.VMEM((rpw, 128), jnp.float32),
                       pltpu.SemaphoreType.DMA,
                       pltpu.SemaphoreType.DMA],
    )
    def _gather_k(x_hbm, sg_hbm, idx_hbm, xg_hbm, sgg_hbm,
                  idx_v, rows_v, sgv, sem1, sem2):
        wid = lax.axis_index("s") * _NC + lax.axis_index("c")
        base = wid * rpw
        pltpu.sync_copy(idx_hbm.at[pl.ds(base, rpw)], idx_v)
        c1 = pltpu.async_copy(x_hbm.at[idx_v], rows_v, sem1)
        c2 = pltpu.async_copy(sg_hbm.at[idx_v], sgv, sem2)
        c1.wait()
        c2.wait()
        pltpu.sync_copy(rows_v, xg_hbm.at[pl.ds(base, rpw)])
        pltpu.sync_copy(sgv, sgg_hbm.at[pl.ds(base, rpw)])

    return _gather_k(xf, sg, gidx)


def _mlp_body(xg_ref, w1_ref, b1_ref, w2_ref, b2_ref, sgg_ref, o_ref):
    j = pl.program_id(0)
    h = jnp.dot(xg_ref[...].astype(jnp.bfloat16), w1_ref[...],
                preferred_element_type=jnp.float32) + b1_ref[...]
    h = jax.nn.gelu(h)
    part = jnp.dot(h.astype(jnp.bfloat16), w2_ref[...],
                   preferred_element_type=jnp.float32)

    @pl.when(j == 0)
    def _():
        o_ref[...] = part

    @pl.when(j > 0)
    def _():
        o_ref[...] += part

    @pl.when(j == pl.num_programs(0) - 1)
    def _():
        scale = sgg_ref[:, 0:1]
        gate = sgg_ref[:, 1:2]
        o_ref[...] = ((o_ref[...] + b2_ref[...]) * scale
                      + xg_ref[...] * (1.0 - gate))


def _mlp(xg, w1, b1r, w2, b2r, sgg):
    bk, d = xg.shape
    dff = w1.shape[1]
    tk = 512
    return pl.pallas_call(
        _mlp_body,
        grid=(dff // tk,),
        in_specs=[
            pl.BlockSpec((bk, d), lambda j: (0, 0)),
            pl.BlockSpec((d, tk), lambda j: (0, j)),
            pl.BlockSpec((1, tk), lambda j: (0, j)),
            pl.BlockSpec((tk, d), lambda j: (j, 0)),
            pl.BlockSpec((1, d), lambda j: (0, 0)),
            pl.BlockSpec((bk, 128), lambda j: (0, 0)),
        ],
        out_specs=pl.BlockSpec((bk, d), lambda j: (0, 0)),
        out_shape=jax.ShapeDtypeStruct((bk, d), jnp.float32),
    )(xg, w1.astype(jnp.bfloat16), b1r, w2.astype(jnp.bfloat16), b2r, sgg)


_SCATTER_RPS = 64


def _scatter_body(idx_ref, x_ref, v_ref, o_ref, sem):
    del x_ref
    i = pl.program_id(0)
    base = i * _SCATTER_RPS
    copies = []
    for r in range(_SCATTER_RPS):
        row = idx_ref[base + r]
        copies.append(pltpu.make_async_copy(
            v_ref.at[pl.ds(r, 1)], o_ref.at[pl.ds(row, 1)], sem))
    for c in copies:
        c.start()
    for c in copies:
        c.wait()


def _scatter(xf, vals, gidx):
    n, d = xf.shape
    bk = gidx.shape[0]
    grid_spec = pltpu.PrefetchScalarGridSpec(
        num_scalar_prefetch=1,
        grid=(bk // _SCATTER_RPS,),
        in_specs=[
            pl.BlockSpec(memory_space=pl.ANY),
            pl.BlockSpec((_SCATTER_RPS, d), lambda i, idx: (i, 0)),
        ],
        out_specs=pl.BlockSpec(memory_space=pl.ANY),
        scratch_shapes=[pltpu.SemaphoreType.DMA],
    )
    return pl.pallas_call(
        _scatter_body,
        grid_spec=grid_spec,
        out_shape=jax.ShapeDtypeStruct((n, d), jnp.float32),
        input_output_aliases={1: 0},
    )(gidx, xf, vals)


def kernel(x, causal_mask, position_ids, router_w, router_b, w1, b1, w2, b2):
    b, s, d = x.shape
    dff = w1.shape[1]
    k = s // 8
    xf = x.reshape(b * s, d)

    wts = _router(xf, router_w)
    idx3, scale3, gate3 = _route(wts.reshape(b, s, 1),
                                 router_b.reshape(1, 1), k)
    gidx = idx3.reshape(b * k)

    sg = jnp.concatenate(
        [scale3.reshape(b * s, 1), gate3.reshape(b * s, 1),
         jnp.zeros((b * s, 126), jnp.float32)], axis=1)

    xg, sgg = _gather(xf, sg, gidx)
    vals = _mlp(xg, w1, b1.reshape(1, dff), w2, b2.reshape(1, d), sgg)
    out = _scatter(xf, vals, gidx)
    return (out.reshape(b, s, d),)


# speedup vs baseline: 8.1601x; 1.2242x over previous
import functools

import jax
import jax.numpy as jnp
import numpy as np
from jax import lax
from jax.experimental import pallas as pl
from jax.experimental.pallas import tpu as pltpu
from jax.experimental.pallas import tpu_sc as plsc

_NC = 2
_NS = 16
_NW = _NC * _NS

_I32_MIN = np.int32(-2147483648)


def _router_body(x_ref, rw_ref, o_ref):
    o_ref[...] = jnp.dot(x_ref[...], rw_ref[...],
                         preferred_element_type=jnp.float32)


def _router(xf, router_w):
    n, d = xf.shape
    tm = 1024
    return pl.pallas_call(
        _router_body,
        grid=(n // tm,),
        in_specs=[
            pl.BlockSpec((tm, d), lambda i: (i, 0)),
            pl.BlockSpec((d, 1), lambda i: (0, 0)),
        ],
        out_specs=pl.BlockSpec((tm, 1), lambda i: (i, 0)),
        out_shape=jax.ShapeDtypeStruct((n, 1), jnp.float32),
    )(xf, router_w)


def _cumsum_col(v, s):
    sh = 1
    while sh < s:
        v = v + jnp.concatenate(
            [jnp.zeros((sh, 1), jnp.float32), v[:-sh, :]], axis=0)
        sh *= 2
    return v


def _route_body(k, w_ref, rb_ref, idx_ref, scale_ref, gate_ref):
    s = w_ref.shape[1]
    w = w_ref[0] + rb_ref[0, 0]
    bits = lax.bitcast_convert_type(w, jnp.int32)
    key = jnp.where(bits >= 0, bits, jnp.bitwise_xor(~bits, _I32_MIN))

    cnt = jnp.sum(jnp.where(key >= 0, 1, 0))
    cs = jnp.where(cnt >= k, jnp.int32(0), _I32_MIN)
    for t in range(30, -1, -1):
        test = jnp.bitwise_or(cs, jnp.int32(1 << t))
        cnt = jnp.sum(jnp.where(key >= test, 1, 0))
        cs = jnp.where(cnt >= k, test, cs)
    thr_bits = jnp.where(cs >= 0, cs, ~jnp.bitwise_xor(cs, _I32_MIN))
    thr = lax.bitcast_convert_type(thr_bits, jnp.float32)

    gt = (w > thr).astype(jnp.float32)
    eq = (w == thr).astype(jnp.float32)
    m = jnp.sum(gt)
    c2 = _cumsum_col(eq, s)
    fill = eq * jnp.where(m + c2 <= k, 1.0, 0.0)
    mask2 = gt + fill
    pos = _cumsum_col(mask2, s)

    jrow = lax.broadcasted_iota(jnp.int32, (1, k), 1).astype(jnp.float32) + 1.0
    one_hot = jnp.where(pos == jrow, 1.0, 0.0) * mask2
    iota_row = lax.broadcasted_iota(jnp.int32, (1, s), 1).astype(jnp.float32)
    idx_f = jnp.dot(iota_row, one_hot,
                    preferred_element_type=jnp.float32,
                    precision=lax.Precision.HIGHEST)
    b = pl.program_id(0)
    idx_ref[0, ...] = idx_f.astype(jnp.int32) + b * s

    scale_ref[0] = w * gt
    gate_ref[0] = gt


def _route(wts3, rb, k):
    bsz, s, _ = wts3.shape
    return pl.pallas_call(
        functools.partial(_route_body, k),
        grid=(bsz,),
        in_specs=[
            pl.BlockSpec((1, s, 1), lambda b: (b, 0, 0)),
            pl.BlockSpec((1, 1), lambda b: (0, 0)),
        ],
        out_specs=[
            pl.BlockSpec((1, 1, k), lambda b: (b, 0, 0)),
            pl.BlockSpec((1, s, 1), lambda b: (b, 0, 0)),
            pl.BlockSpec((1, s, 1), lambda b: (b, 0, 0)),
        ],
        out_shape=[
            jax.ShapeDtypeStruct((bsz, 1, k), jnp.int32),
            jax.ShapeDtypeStruct((bsz, s, 1), jnp.float32),
            jax.ShapeDtypeStruct((bsz, s, 1), jnp.float32),
        ],
    )(wts3, rb)


def _gather(xf, sg, gidx):
    n, d = xf.shape
    bk = gidx.shape[0]
    rpw = bk // _NW
    mesh = plsc.VectorSubcoreMesh(core_axis_name="c", subcore_axis_name="s",
                                  num_cores=_NC, num_subcores=_NS)

    @functools.partial(
        pl.kernel,
        out_type=[jax.ShapeDtypeStruct((bk, d), jnp.float32),
                  jax.ShapeDtypeStruct((bk, 128), jnp.float32)],
        mesh=mesh,
        scratch_types=[pltpu.VMEM((rpw,), jnp.int32),
                       pltpu.VMEM((rpw, d), jnp.float32),
                       pltpu.VMEM((rpw, 128), jnp.float32),
                       pltpu.SemaphoreType.DMA,
                       pltpu.SemaphoreType.DMA],
    )
    def _gather_k(x_hbm, sg_hbm, idx_hbm, xg_hbm, sgg_hbm,
                  idx_v, rows_v, sgv, sem1, sem2):
        wid = lax.axis_index("s") * _NC + lax.axis_index("c")
        base = wid * rpw
        pltpu.sync_copy(idx_hbm.at[pl.ds(base, rpw)], idx_v)
        c1 = pltpu.async_copy(x_hbm.at[idx_v], rows_v, sem1)
        c2 = pltpu.async_copy(sg_hbm.at[idx_v], sgv, sem2)
        c1.wait()
        c2.wait()
        pltpu.sync_copy(rows_v, xg_hbm.at[pl.ds(base, rpw)])
        pltpu.sync_copy(sgv, sgg_hbm.at[pl.ds(base, rpw)])

    return _gather_k(xf, sg, gidx)


def _mlp_body(xg_ref, w1_ref, b1_ref, w2_ref, b2_ref, sgg_ref, o_ref):
    j = pl.program_id(0)
    h = jnp.dot(xg_ref[...].astype(jnp.bfloat16),
                w1_ref[...].astype(jnp.bfloat16),
                preferred_element_type=jnp.float32) + b1_ref[...]
    h = jax.nn.gelu(h)
    part = jnp.dot(h.astype(jnp.bfloat16), w2_ref[...].astype(jnp.bfloat16),
                   preferred_element_type=jnp.float32)

    @pl.when(j == 0)
    def _():
        o_ref[...] = part

    @pl.when(j > 0)
    def _():
        o_ref[...] += part

    @pl.when(j == pl.num_programs(0) - 1)
    def _():
        scale = sgg_ref[:, 0:1]
        gate = sgg_ref[:, 1:2]
        o_ref[...] = ((o_ref[...] + b2_ref[...]) * scale
                      + xg_ref[...] * (1.0 - gate))


def _mlp(xg, w1, b1r, w2, b2r, sgg):
    bk, d = xg.shape
    dff = w1.shape[1]
    tk = 512
    return pl.pallas_call(
        _mlp_body,
        grid=(dff // tk,),
        in_specs=[
            pl.BlockSpec((bk, d), lambda j: (0, 0)),
            pl.BlockSpec((d, tk), lambda j: (0, j)),
            pl.BlockSpec((1, tk), lambda j: (0, j)),
            pl.BlockSpec((tk, d), lambda j: (j, 0)),
            pl.BlockSpec((1, d), lambda j: (0, 0)),
            pl.BlockSpec((bk, 128), lambda j: (0, 0)),
        ],
        out_specs=pl.BlockSpec((bk, d), lambda j: (0, 0)),
        out_shape=jax.ShapeDtypeStruct((bk, d), jnp.float32),
    )(xg, w1, b1r, w2, b2r, sgg)


_SCATTER_RPS = 64


def _scatter_body(idx_ref, x_ref, v_ref, o_ref, scr, sems):
    i = pl.program_id(0)
    nsteps = pl.num_programs(0)
    par = i % 2
    base = i * _SCATTER_RPS

    def drain(p):
        pltpu.make_async_copy(x_ref.at[pl.ds(0, _SCATTER_RPS)],
                              scr.at[p], sems.at[p]).wait()

    @pl.when(i >= 2)
    def _():
        drain(par)

    scr[par] = v_ref[...]
    for r in range(_SCATTER_RPS):
        row = idx_ref[base + r]
        pltpu.make_async_copy(scr.at[par, pl.ds(r, 1)],
                              o_ref.at[pl.ds(row, 1)], sems.at[par]).start()

    @pl.when(i == nsteps - 1)
    def _():
        drain(1 - par)
        drain(par)


def _scatter(xf, vals, gidx):
    n, d = xf.shape
    bk = gidx.shape[0]
    grid_spec = pltpu.PrefetchScalarGridSpec(
        num_scalar_prefetch=1,
        grid=(bk // _SCATTER_RPS,),
        in_specs=[
            pl.BlockSpec(memory_space=pl.ANY),
            pl.BlockSpec((_SCATTER_RPS, d), lambda i, idx: (i, 0)),
        ],
        out_specs=pl.BlockSpec(memory_space=pl.ANY),
        scratch_shapes=[
            pltpu.VMEM((2, _SCATTER_RPS, d), jnp.float32),
            pltpu.SemaphoreType.DMA((2,)),
        ],
    )
    return pl.pallas_call(
        _scatter_body,
        grid_spec=grid_spec,
        out_shape=jax.ShapeDtypeStruct((n, d), jnp.float32),
        input_output_aliases={1: 0},
    )(gidx, xf, vals)


def kernel(x, causal_mask, position_ids, router_w, router_b, w1, b1, w2, b2):
    b, s, d = x.shape
    dff = w1.shape[1]
    k = s // 8
    xf = x.reshape(b * s, d)

    wts = _router(xf, router_w)
    idx3, scale3, gate3 = _route(wts.reshape(b, s, 1),
                                 router_b.reshape(1, 1), k)
    gidx = idx3.reshape(b * k)

    sg = jnp.concatenate(
        [scale3.reshape(b * s, 1), gate3.reshape(b * s, 1),
         jnp.zeros((b * s, 126), jnp.float32)], axis=1)

    xg, sgg = _gather(xf, sg, gidx)
    vals = _mlp(xg, w1, b1.reshape(1, dff), w2, b2.reshape(1, d), sgg)
    out = _scatter(xf, vals, gidx)
    return (out.reshape(b, s, d),)


# speedup vs baseline: 8.1847x; 1.0030x over previous
import functools

import jax
import jax.numpy as jnp
import numpy as np
from jax import lax
from jax.experimental import pallas as pl
from jax.experimental.pallas import tpu as pltpu
from jax.experimental.pallas import tpu_sc as plsc

_NC = 2
_NS = 16
_NW = _NC * _NS

_I32_MIN = np.int32(-2147483648)


def _router_body(x_ref, rw_ref, o_ref):
    o_ref[...] = jnp.dot(x_ref[...], rw_ref[...],
                         preferred_element_type=jnp.float32)


def _router(xf, router_w):
    n, d = xf.shape
    tm = 1024
    rw_pad = jnp.pad(router_w, ((0, 0), (0, 127)))
    out = pl.pallas_call(
        _router_body,
        grid=(n // tm,),
        in_specs=[
            pl.BlockSpec((tm, d), lambda i: (i, 0)),
            pl.BlockSpec((d, 128), lambda i: (0, 0)),
        ],
        out_specs=pl.BlockSpec((tm, 128), lambda i: (i, 0)),
        out_shape=jax.ShapeDtypeStruct((n, 128), jnp.float32),
    )(xf, rw_pad)
    return out[:, 0:1]


def _cumsum_col(v, s):
    sh = 1
    while sh < s:
        v = v + jnp.concatenate(
            [jnp.zeros((sh, 1), jnp.float32), v[:-sh, :]], axis=0)
        sh *= 2
    return v


def _route_body(k, w_ref, rb_ref, idx_ref, scale_ref, gate_ref):
    s = w_ref.shape[1]
    w = w_ref[0] + rb_ref[0, 0]
    bits = lax.bitcast_convert_type(w, jnp.int32)
    key = jnp.where(bits >= 0, bits, jnp.bitwise_xor(~bits, _I32_MIN))

    cnt = jnp.sum(jnp.where(key >= 0, 1, 0))
    cs = jnp.where(cnt >= k, jnp.int32(0), _I32_MIN)
    for t in range(30, -1, -1):
        test = jnp.bitwise_or(cs, jnp.int32(1 << t))
        cnt = jnp.sum(jnp.where(key >= test, 1, 0))
        cs = jnp.where(cnt >= k, test, cs)
    thr_bits = jnp.where(cs >= 0, cs, ~jnp.bitwise_xor(cs, _I32_MIN))
    thr = lax.bitcast_convert_type(thr_bits, jnp.float32)

    gt = (w > thr).astype(jnp.float32)
    eq = (w == thr).astype(jnp.float32)
    m = jnp.sum(gt)
    c2 = _cumsum_col(eq, s)
    fill = eq * jnp.where(m + c2 <= k, 1.0, 0.0)
    mask2 = gt + fill
    pos = _cumsum_col(mask2, s)

    jrow = lax.broadcasted_iota(jnp.int32, (1, k), 1).astype(jnp.float32) + 1.0
    one_hot = jnp.where(pos == jrow, 1.0, 0.0) * mask2
    iota_row = lax.broadcasted_iota(jnp.int32, (1, s), 1).astype(jnp.float32)
    idx_f = jnp.dot(iota_row, one_hot,
                    preferred_element_type=jnp.float32,
                    precision=lax.Precision.HIGHEST)
    b = pl.program_id(0)
    idx_ref[0, ...] = idx_f.astype(jnp.int32) + b * s

    scale_ref[0] = w * gt
    gate_ref[0] = gt


def _route(wts3, rb, k):
    bsz, s, _ = wts3.shape
    return pl.pallas_call(
        functools.partial(_route_body, k),
        grid=(bsz,),
        in_specs=[
            pl.BlockSpec((1, s, 1), lambda b: (b, 0, 0)),
            pl.BlockSpec((1, 1), lambda b: (0, 0)),
        ],
        out_specs=[
            pl.BlockSpec((1, 1, k), lambda b: (b, 0, 0)),
            pl.BlockSpec((1, s, 1), lambda b: (b, 0, 0)),
            pl.BlockSpec((1, s, 1), lambda b: (b, 0, 0)),
        ],
        out_shape=[
            jax.ShapeDtypeStruct((bsz, 1, k), jnp.int32),
            jax.ShapeDtypeStruct((bsz, s, 1), jnp.float32),
            jax.ShapeDtypeStruct((bsz, s, 1), jnp.float32),
        ],
    )(wts3, rb)


def _gather(xf, sg, gidx):
    n, d = xf.shape
    bk = gidx.shape[0]
    rpw = bk // _NW
    mesh = plsc.VectorSubcoreMesh(core_axis_name="c", subcore_axis_name="s",
                                  num_cores=_NC, num_subcores=_NS)

    @functools.partial(
        pl.kernel,
        out_type=[jax.ShapeDtypeStruct((bk, d), jnp.float32),
                  jax.ShapeDtypeStruct((bk, 128), jnp.float32)],
        mesh=mesh,
        scratch_types=[pltpu.VMEM((rpw,), jnp.int32),
                       pltpu.VMEM((rpw, d), jnp.float32),
                       pltpu.VMEM((rpw, 128), jnp.float32),
                       pltpu.SemaphoreType.DMA,
                       pltpu.SemaphoreType.DMA],
    )
    def _gather_k(x_hbm, sg_hbm, idx_hbm, xg_hbm, sgg_hbm,
                  idx_v, rows_v, sgv, sem1, sem2):
        wid = lax.axis_index("s") * _NC + lax.axis_index("c")
        base = wid * rpw
        pltpu.sync_copy(idx_hbm.at[pl.ds(base, rpw)], idx_v)
        c1 = pltpu.async_copy(x_hbm.at[idx_v], rows_v, sem1)
        c2 = pltpu.async_copy(sg_hbm.at[idx_v], sgv, sem2)
        c1.wait()
        c2.wait()
        pltpu.sync_copy(rows_v, xg_hbm.at[pl.ds(base, rpw)])
        pltpu.sync_copy(sgv, sgg_hbm.at[pl.ds(base, rpw)])

    return _gather_k(xf, sg, gidx)


def _mlp_body(xg_ref, w1_ref, b1_ref, w2_ref, b2_ref, sgg_ref, o_ref):
    j = pl.program_id(0)
    h = jnp.dot(xg_ref[...].astype(jnp.bfloat16),
                w1_ref[...].astype(jnp.bfloat16),
                preferred_element_type=jnp.float32) + b1_ref[...]
    h = jax.nn.gelu(h)
    part = jnp.dot(h.astype(jnp.bfloat16), w2_ref[...].astype(jnp.bfloat16),
                   preferred_element_type=jnp.float32)

    @pl.when(j == 0)
    def _():
        o_ref[...] = part

    @pl.when(j > 0)
    def _():
        o_ref[...] += part

    @pl.when(j == pl.num_programs(0) - 1)
    def _():
        scale = sgg_ref[:, 0:1]
        gate = sgg_ref[:, 1:2]
        o_ref[...] = ((o_ref[...] + b2_ref[...]) * scale
                      + xg_ref[...] * (1.0 - gate))


def _mlp(xg, w1, b1r, w2, b2r, sgg):
    bk, d = xg.shape
    dff = w1.shape[1]
    tk = 512
    return pl.pallas_call(
        _mlp_body,
        grid=(dff // tk,),
        in_specs=[
            pl.BlockSpec((bk, d), lambda j: (0, 0)),
            pl.BlockSpec((d, tk), lambda j: (0, j)),
            pl.BlockSpec((1, tk), lambda j: (0, j)),
            pl.BlockSpec((tk, d), lambda j: (j, 0)),
            pl.BlockSpec((1, d), lambda j: (0, 0)),
            pl.BlockSpec((bk, 128), lambda j: (0, 0)),
        ],
        out_specs=pl.BlockSpec((bk, d), lambda j: (0, 0)),
        out_shape=jax.ShapeDtypeStruct((bk, d), jnp.float32),
    )(xg, w1, b1r, w2, b2r, sgg)


_SCATTER_RPS = 64


def _scatter_body(idx_ref, x_ref, v_ref, o_ref, scr, sems):
    i = pl.program_id(0)
    nsteps = pl.num_programs(0)
    par = i % 2
    base = i * _SCATTER_RPS

    def drain(p):
        pltpu.make_async_copy(x_ref.at[pl.ds(0, _SCATTER_RPS)],
                              scr.at[p], sems.at[p]).wait()

    @pl.when(i >= 2)
    def _():
        drain(par)

    scr[par] = v_ref[...]
    for r in range(_SCATTER_RPS):
        row = idx_ref[base + r]
        pltpu.make_async_copy(scr.at[par, pl.ds(r, 1)],
                              o_ref.at[pl.ds(row, 1)], sems.at[par]).start()

    @pl.when(i == nsteps - 1)
    def _():
        drain(1 - par)
        drain(par)


def _scatter(xf, vals, gidx):
    n, d = xf.shape
    bk = gidx.shape[0]
    grid_spec = pltpu.PrefetchScalarGridSpec(
        num_scalar_prefetch=1,
        grid=(bk // _SCATTER_RPS,),
        in_specs=[
            pl.BlockSpec(memory_space=pl.ANY),
            pl.BlockSpec((_SCATTER_RPS, d), lambda i, idx: (i, 0)),
        ],
        out_specs=pl.BlockSpec(memory_space=pl.ANY),
        scratch_shapes=[
            pltpu.VMEM((2, _SCATTER_RPS, d), jnp.float32),
            pltpu.SemaphoreType.DMA((2,)),
        ],
    )
    return pl.pallas_call(
        _scatter_body,
        grid_spec=grid_spec,
        out_shape=jax.ShapeDtypeStruct((n, d), jnp.float32),
        input_output_aliases={1: 0},
    )(gidx, xf, vals)


def kernel(x, causal_mask, position_ids, router_w, router_b, w1, b1, w2, b2):
    b, s, d = x.shape
    dff = w1.shape[1]
    k = s // 8
    xf = x.reshape(b * s, d)

    wts = _router(xf, router_w)
    idx3, scale3, gate3 = _route(wts.reshape(b, s, 1),
                                 router_b.reshape(1, 1), k)
    gidx = idx3.reshape(b * k)

    sg = jnp.concatenate(
        [scale3.reshape(b * s, 1), gate3.reshape(b * s, 1),
         jnp.zeros((b * s, 126), jnp.float32)], axis=1)

    xg, sgg = _gather(xf, sg, gidx)
    vals = _mlp(xg, w1, b1.reshape(1, dff), w2, b2.reshape(1, d), sgg)
    out = _scatter(xf, vals, gidx)
    return (out.reshape(b, s, d),)


# speedup vs baseline: 9.7043x; 1.1857x over previous
import functools

import jax
import jax.numpy as jnp
import numpy as np
from jax import lax
from jax.experimental import pallas as pl
from jax.experimental.pallas import tpu as pltpu
from jax.experimental.pallas import tpu_sc as plsc

_NC = 2
_NS = 16
_NW = _NC * _NS

_I32_MIN = np.int32(-2147483648)


def _router_body(x_ref, rw_ref, o_ref):
    o_ref[...] = jnp.dot(x_ref[...], rw_ref[...],
                         preferred_element_type=jnp.float32)


def _router(xf, router_w):
    n, d = xf.shape
    tm = 1024
    rw_pad = jnp.pad(router_w, ((0, 0), (0, 127)))
    out = pl.pallas_call(
        _router_body,
        grid=(n // tm,),
        in_specs=[
            pl.BlockSpec((tm, d), lambda i: (i, 0)),
            pl.BlockSpec((d, 128), lambda i: (0, 0)),
        ],
        out_specs=pl.BlockSpec((tm, 128), lambda i: (i, 0)),
        out_shape=jax.ShapeDtypeStruct((n, 128), jnp.float32),
    )(xf, rw_pad)
    return out[:, 0:1]


def _cumsum_cols(v, s):
    sh = 1
    while sh < s:
        v = v + jnp.concatenate(
            [jnp.zeros((sh, v.shape[1]), jnp.float32), v[:-sh, :]], axis=0)
        sh *= 2
    return v


def _route_body(k, wr_ref, wc_ref, rb_ref, idx_ref, sidx_ref, scale_ref):
    nb, s = wr_ref.shape
    rb = rb_ref[0, 0]
    w_r = wr_ref[...] + rb
    bits = lax.bitcast_convert_type(w_r, jnp.int32)
    key = jnp.where(bits >= 0, bits, jnp.bitwise_xor(~bits, _I32_MIN))

    cnt = jnp.sum(jnp.where(key >= 0, 1, 0), axis=1, keepdims=True)
    cs = jnp.where(cnt >= k, jnp.int32(0), _I32_MIN)
    for t in range(30, -1, -1):
        test = jnp.bitwise_or(cs, jnp.int32(1 << t))
        cnt = jnp.sum(jnp.where(key >= test, 1, 0), axis=1, keepdims=True)
        cs = jnp.where(cnt >= k, test, cs)
    thr_bits = jnp.where(cs >= 0, cs, ~jnp.bitwise_xor(cs, _I32_MIN))
    thr = lax.bitcast_convert_type(thr_bits, jnp.float32)
    gt_r = (w_r > thr).astype(jnp.float32)

    eyeb = jnp.where(lax.broadcasted_iota(jnp.int32, (nb, nb), 0)
                     == lax.broadcasted_iota(jnp.int32, (nb, nb), 1), 1.0, 0.0)
    thr_c = jnp.sum(thr * eyeb, axis=0, keepdims=True)
    w_c = wc_ref[...] + rb
    gt_c = (w_c > thr_c).astype(jnp.float32)
    eq_c = (w_c == thr_c).astype(jnp.float32)
    m_c = jnp.sum(gt_c, axis=0, keepdims=True)
    c2 = _cumsum_cols(eq_c, s)
    fill = eq_c * jnp.where(m_c + c2 <= k, 1.0, 0.0)
    mask2 = gt_c + fill
    pos = _cumsum_cols(mask2, s)

    jrow = lax.broadcasted_iota(jnp.int32, (1, k), 1).astype(jnp.float32) + 1.0
    iota_row = lax.broadcasted_iota(jnp.int32, (1, s), 1).astype(jnp.float32)
    for b in range(nb):
        one_hot = (jnp.where(pos[:, b:b + 1] == jrow, 1.0, 0.0)
                   * mask2[:, b:b + 1])
        idx_f = jnp.dot(iota_row, one_hot,
                        preferred_element_type=jnp.float32,
                        precision=lax.Precision.HIGHEST)
        wsel = jnp.dot(w_r[b:b + 1, :], one_hot,
                       preferred_element_type=jnp.float32,
                       precision=lax.Precision.HIGHEST)
        gtk = jnp.dot(gt_r[b:b + 1, :], one_hot,
                      preferred_element_type=jnp.float32,
                      precision=lax.Precision.HIGHEST)
        idx_i = idx_f.astype(jnp.int32) + b * s
        idx_ref[b] = idx_i
        sidx_ref[b] = jnp.where(gtk >= 0.5, idx_i, idx_i + nb * s)
        scale_ref[b] = wsel


def _route(wts2, rb, k):
    bsz, s = wts2.shape
    return pl.pallas_call(
        functools.partial(_route_body, k),
        in_specs=[
            pl.BlockSpec((bsz, s), lambda: (0, 0)),
            pl.BlockSpec((s, bsz), lambda: (0, 0)),
            pl.BlockSpec((1, 1), lambda: (0, 0)),
        ],
        out_specs=[
            pl.BlockSpec((bsz, 1, k), lambda: (0, 0, 0)),
            pl.BlockSpec((bsz, 1, k), lambda: (0, 0, 0)),
            pl.BlockSpec((bsz, 1, k), lambda: (0, 0, 0)),
        ],
        out_shape=[
            jax.ShapeDtypeStruct((bsz, 1, k), jnp.int32),
            jax.ShapeDtypeStruct((bsz, 1, k), jnp.int32),
            jax.ShapeDtypeStruct((bsz, 1, k), jnp.float32),
        ],
    )(wts2, wts2.T, rb)


def _gather(xf, gidx):
    n, d = xf.shape
    bk = gidx.shape[0]
    rpw = bk // _NW
    mesh = plsc.VectorSubcoreMesh(core_axis_name="c", subcore_axis_name="s",
                                  num_cores=_NC, num_subcores=_NS)

    @functools.partial(
        pl.kernel,
        out_type=jax.ShapeDtypeStruct((bk, d), jnp.float32),
        mesh=mesh,
        scratch_types=[pltpu.VMEM((rpw,), jnp.int32),
                       pltpu.VMEM((rpw, d), jnp.float32),
                       pltpu.SemaphoreType.DMA],
    )
    def _gather_k(x_hbm, idx_hbm, xg_hbm, idx_v, rows_v, sem):
        wid = lax.axis_index("s") * _NC + lax.axis_index("c")
        base = wid * rpw
        pltpu.sync_copy(idx_hbm.at[pl.ds(base, rpw)], idx_v)
        pltpu.async_copy(x_hbm.at[idx_v], rows_v, sem).wait()
        pltpu.sync_copy(rows_v, xg_hbm.at[pl.ds(base, rpw)])

    return _gather_k(xf, gidx)


def _mlp_body(xg_ref, w1_ref, b1_ref, w2_ref, b2_ref, sc_ref, o_ref):
    j = pl.program_id(0)
    h = jnp.dot(xg_ref[...], w1_ref[...],
                preferred_element_type=jnp.float32) + b1_ref[...]
    h = jax.nn.gelu(h)
    part = jnp.dot(h, w2_ref[...], preferred_element_type=jnp.float32)

    @pl.when(j == 0)
    def _():
        o_ref[...] = part

    @pl.when(j > 0)
    def _():
        o_ref[...] += part

    @pl.when(j == pl.num_programs(0) - 1)
    def _():
        o_ref[...] = (o_ref[...] + b2_ref[...]) * sc_ref[...]


def _mlp(xg, w1, b1r, w2, b2r, scale2):
    bk, d = xg.shape
    dff = w1.shape[1]
    tk = 512
    return pl.pallas_call(
        _mlp_body,
        grid=(dff // tk,),
        in_specs=[
            pl.BlockSpec((bk, d), lambda j: (0, 0)),
            pl.BlockSpec((d, tk), lambda j: (0, j)),
            pl.BlockSpec((1, tk), lambda j: (0, j)),
            pl.BlockSpec((tk, d), lambda j: (j, 0)),
            pl.BlockSpec((1, d), lambda j: (0, 0)),
            pl.BlockSpec((bk, 1), lambda j: (0, 0)),
        ],
        out_specs=pl.BlockSpec((bk, d), lambda j: (0, 0)),
        out_shape=jax.ShapeDtypeStruct((bk, d), jnp.float32),
    )(xg, w1, b1r, w2, b2r, scale2)


_SCATTER_RPS = 64


def _scatter_body(idx_ref, x_ref, v_ref, o_ref, scr, sems):
    i = pl.program_id(0)
    nsteps = pl.num_programs(0)
    n = x_ref.shape[0]
    par = i % 2
    base = i * _SCATTER_RPS

    def drain(p):
        pltpu.make_async_copy(x_ref.at[pl.ds(0, _SCATTER_RPS)],
                              scr.at[p], sems.at[p]).wait()

    @pl.when(i >= 2)
    def _():
        drain(par)

    scr[par] = v_ref[...]
    for r in range(_SCATTER_RPS):
        raw = idx_ref[base + r]
        row = lax.rem(raw, n)

        @pl.when(raw < n)
        def _():
            pltpu.make_async_copy(scr.at[par, pl.ds(r, 1)],
                                  o_ref.at[pl.ds(row, 1)],
                                  sems.at[par]).start()

        @pl.when(raw >= n)
        def _():
            pltpu.make_async_copy(x_ref.at[pl.ds(row, 1)],
                                  o_ref.at[pl.ds(row, 1)],
                                  sems.at[par]).start()

    @pl.when(i == nsteps - 1)
    def _():
        drain(1 - par)
        drain(par)


def _scatter(xf, vals, sidx):
    n, d = xf.shape
    bk = sidx.shape[0]
    grid_spec = pltpu.PrefetchScalarGridSpec(
        num_scalar_prefetch=1,
        grid=(bk // _SCATTER_RPS,),
        in_specs=[
            pl.BlockSpec(memory_space=pl.ANY),
            pl.BlockSpec((_SCATTER_RPS, d), lambda i, idx: (i, 0)),
        ],
        out_specs=pl.BlockSpec(memory_space=pl.ANY),
        scratch_shapes=[
            pltpu.VMEM((2, _SCATTER_RPS, d), jnp.float32),
            pltpu.SemaphoreType.DMA((2,)),
        ],
    )
    return pl.pallas_call(
        _scatter_body,
        grid_spec=grid_spec,
        out_shape=jax.ShapeDtypeStruct((n, d), jnp.float32),
        input_output_aliases={1: 0},
    )(sidx, xf, vals)


def kernel(x, causal_mask, position_ids, router_w, router_b, w1, b1, w2, b2):
    b, s, d = x.shape
    dff = w1.shape[1]
    k = s // 8
    xf = x.reshape(b * s, d)

    wts = _router(xf, router_w)
    idx3, sidx3, scale3 = _route(wts.reshape(b, s), router_b.reshape(1, 1), k)
    gidx = idx3.reshape(b * k)
    sidx = sidx3.reshape(b * k)

    xg = _gather(xf, gidx)
    vals = _mlp(xg, w1, b1.reshape(1, dff), w2, b2.reshape(1, d),
                scale3.reshape(b * k, 1))
    out = _scatter(xf, vals, sidx)
    return (out.reshape(b, s, d),)
